# baseline (speedup 1.0000x reference)
import functools

import jax
import jax.numpy as jnp
from jax import lax
from jax.experimental import pallas as pl
from jax.experimental.pallas import tpu as pltpu
from jax.experimental.pallas import tpu_sc as plsc

LANES = 16
NW = 32


def _dense_relu_kernel(x_ref, w_ref, b_ref, o_ref):
    acc = jnp.dot(x_ref[...], w_ref[...], preferred_element_type=jnp.float32)
    o_ref[...] = jnp.maximum(acc + b_ref[...], 0.0)


def _dense_relu(x, W, b, block_rows):
    n, d = x.shape
    h = W.shape[1]
    assert n % block_rows == 0
    return pl.pallas_call(
        _dense_relu_kernel,
        grid=(n // block_rows,),
        in_specs=[
            pl.BlockSpec((block_rows, d), lambda i: (i, 0)),
            pl.BlockSpec((d, h), lambda i: (0, 0)),
            pl.BlockSpec((1, h), lambda i: (0, 0)),
        ],
        out_specs=pl.BlockSpec((block_rows, h), lambda i: (i, 0)),
        out_shape=jax.ShapeDtypeStruct((n, h), jnp.float32),
    )(x, W, b.reshape(1, h))


def _make_accumulate(n_pad, K, H, per_w, C):
    assert per_w * NW == n_pad and per_w % (2 * C) == 0
    n_chunks = per_w // C
    n_pairs = n_chunks // 2
    G = (C * K) // 128
    assert G * 128 == C * K and G % 2 == 0
    GH = G // 2
    CH = C // 2
    HV = H // LANES
    mesh = plsc.VectorSubcoreMesh(core_axis_name="c", subcore_axis_name="s")

    @functools.partial(
        pl.kernel,
        out_type=jax.ShapeDtypeStruct((n_pad, 2 * H), jnp.float32),
        mesh=mesh,
        compiler_params=pltpu.CompilerParams(use_tc_tiling_on_sc=False),
        scratch_types=[
            [pltpu.VMEM((G, 128), jnp.int32) for _ in range(2)],
            [pltpu.VMEM((C * K,), jnp.float32) for _ in range(2)],
            [pltpu.VMEM((C, H), jnp.float32) for _ in range(2)],
            pltpu.VMEM((C * K, H), jnp.float32),
            [pltpu.VMEM((C, 2 * H), jnp.float32) for _ in range(2)],
            [pltpu.SemaphoreType.DMA for _ in range(2)],
            [pltpu.SemaphoreType.DMA for _ in range(2)],
            [pltpu.SemaphoreType.DMA for _ in range(2)],
        ],
    )
    def acc(h_hbm, idx_hbm, dsq_hbm, out_hbm,
            idx_v, dsq_v, own_v, rows_v, out_v, lin_s, gat_s, out_s):
        wid = lax.axis_index("s") * 2 + lax.axis_index("c")
        base0 = wid * per_w

        def lin_descs(c, b):
            base = base0 + c * C
            return (
                (idx_hbm.at[pl.ds(base * K // 128, G)], idx_v[b]),
                (dsq_hbm.at[pl.ds(base * K, C * K)], dsq_v[b]),
                (h_hbm.at[pl.ds(base, C)], own_v[b]),
            )

        def issue_linear(c, b):
            for src, dst in lin_descs(c, b):
                pltpu.async_copy(src, dst, lin_s[b])

        def wait_linear(c, b):
            for src, dst in lin_descs(c, b):
                pltpu.make_async_copy(src, dst, lin_s[b]).wait()

        def issue_gathers(b):
            for g in range(G):
                pltpu.async_copy(h_hbm.at[idx_v[b].at[g]],
                                 rows_v.at[pl.ds(g * 128, 128)],
                                 gat_s[g // GH])

        def wait_gather_half(b, half):
            for g in range(half * GH, (half + 1) * GH):
                pltpu.make_async_copy(h_hbm.at[idx_v[b].at[g]],
                                      rows_v.at[pl.ds(g * 128, 128)],
                                      gat_s[half]).wait()

        def out_desc(c, b):
            return (out_v[b], out_hbm.at[pl.ds(base0 + c * C, C)])

        def compute_half(b, half):
            def node_body(i, carry2):
                wvec = jnp.exp(dsq_v[b][pl.ds(i * K, K)] * (-10.0))
                rb = i * K
                s = [jnp.zeros((LANES,), jnp.float32) for _ in range(HV)]
                m = [jnp.full((LANES,), -jnp.inf, jnp.float32)
                     for _ in range(HV)]
                for k in range(K):
                    wk = wvec[k]
                    for j in range(HV):
                        wg = rows_v[rb + k, pl.ds(j * LANES, LANES)] * wk
                        s[j] = s[j] + wg
                        m[j] = jnp.maximum(m[j], wg)
                for j in range(HV):
                    o = own_v[b][i, pl.ds(j * LANES, LANES)]
                    out_v[b][i, pl.ds(j * LANES, LANES)] = s[j] * (1.0 / K) - o
                    out_v[b][i, pl.ds(H + j * LANES, LANES)] = m[j] - o
                return carry2

            lax.fori_loop(half * CH, (half + 1) * CH, node_body, 0)

        issue_linear(0, 0)

        def pair_body(p, carry):
            for b in range(2):
                c = 2 * p + b
                wait_linear(c, b)
                issue_gathers(b)
                if b == 0:
                    issue_linear(c + 1, 1)
                else:
                    @pl.when(p < n_pairs - 1)
                    def _():
                        issue_linear(c + 1, 0)
                wait_gather_half(b, 0)

                @pl.when(p > 0)
                def _():
                    src, dst = out_desc(c - 2, b)
                    pltpu.make_async_copy(src, dst, out_s[b]).wait()

                compute_half(b, 0)
                wait_gather_half(b, 1)
                compute_half(b, 1)
                src, dst = out_desc(c, b)
                pltpu.async_copy(src, dst, out_s[b])
            return carry

        lax.fori_loop(0, n_pairs, pair_body, 0)
        for b in range(2):
            src, dst = out_desc(n_chunks - 2 + b, b)
            pltpu.make_async_copy(src, dst, out_s[b]).wait()

    return acc


def kernel(x, neighbor_indices, distancesq, W0, b0, W1, b1):
    n, d = x.shape
    K = neighbor_indices.shape[1]
    H = W0.shape[1]

    C = 64
    per_w = -(-n // (NW * 2 * C)) * 2 * C
    n_pad = per_w * NW

    pad_n = n_pad - n
    x_pad = jnp.pad(x, ((0, pad_n), (0, 0)))
    idx2d = jnp.pad(neighbor_indices, ((0, pad_n), (0, 0))).reshape(-1, 128)
    dsq_flat = jnp.pad(distancesq, ((0, pad_n), (0, 0))).reshape(-1)

    acc = _make_accumulate(n_pad, K, H, per_w, C)

    h0 = _dense_relu(x_pad, W0, b0, block_rows=512)
    f1 = acc(h0, idx2d, dsq_flat)
    h1 = _dense_relu(f1, W1, b1, block_rows=512)
    f2 = acc(h1, idx2d, dsq_flat)
    return jnp.concatenate([f1[:n], f2[:n], x], axis=-1)


# speedup vs baseline: 1.8188x; 1.8188x over previous
import functools

import jax
import jax.numpy as jnp
from jax import lax
from jax.experimental import pallas as pl
from jax.experimental.pallas import tpu as pltpu
from jax.experimental.pallas import tpu_sc as plsc

LANES = 16
NW = 32


def _dense_relu_kernel(x_ref, w_ref, b_ref, o_ref):
    acc = jnp.dot(x_ref[...], w_ref[...], preferred_element_type=jnp.float32)
    o_ref[...] = jnp.maximum(acc + b_ref[...], 0.0)


def _dense_relu(x, W, b, block_rows):
    n, d = x.shape
    h = W.shape[1]
    assert n % block_rows == 0
    return pl.pallas_call(
        _dense_relu_kernel,
        grid=(n // block_rows,),
        in_specs=[
            pl.BlockSpec((block_rows, d), lambda i: (i, 0)),
            pl.BlockSpec((d, h), lambda i: (0, 0)),
            pl.BlockSpec((1, h), lambda i: (0, 0)),
        ],
        out_specs=pl.BlockSpec((block_rows, h), lambda i: (i, 0)),
        out_shape=jax.ShapeDtypeStruct((n, h), jnp.float32),
    )(x, W, b.reshape(1, h))


def _make_accumulate(n_pad, K, H, per_w, C):
    assert per_w * NW == n_pad and per_w % (2 * C) == 0
    n_chunks = per_w // C
    CK = C * K
    G = CK // 128
    assert G * 128 == CK
    HV = H // LANES
    mesh = plsc.VectorSubcoreMesh(core_axis_name="c", subcore_axis_name="s")

    @functools.partial(
        pl.kernel,
        out_type=jax.ShapeDtypeStruct((n_pad, 2 * H), jnp.float32),
        mesh=mesh,
        compiler_params=pltpu.CompilerParams(use_tc_tiling_on_sc=False),
        scratch_types=[
            pltpu.VMEM((3 * CK,), jnp.int32),
            pltpu.VMEM((3 * CK,), jnp.float32),
            pltpu.VMEM((3 * C, H), jnp.float32),
            pltpu.VMEM((2 * CK, H), jnp.float32),
            pltpu.VMEM((2 * C, 2 * H), jnp.float32),
            pltpu.SemaphoreType.DMA,
            pltpu.SemaphoreType.DMA,
            pltpu.SemaphoreType.DMA,
            pltpu.SemaphoreType.DMA,
        ],
    )
    def acc(h_hbm, idx_hbm, dsq_hbm, out_hbm,
            idx_v, dsq_v, own_v, rows_v, out_v, lin_s, gat_s, out_s0, out_s1):
        wid = lax.axis_index("s") * 2 + lax.axis_index("c")
        base0 = wid * per_w

        def lin_descs(c):
            base = base0 + c * C
            o = (c % 3) * CK
            oc = (c % 3) * C
            return (
                (idx_hbm.at[pl.ds(base * K, CK)], idx_v.at[pl.ds(o, CK)]),
                (dsq_hbm.at[pl.ds(base * K, CK)], dsq_v.at[pl.ds(o, CK)]),
                (h_hbm.at[pl.ds(base, C)], own_v.at[pl.ds(oc, C)]),
            )

        def issue_linear(c):
            for src, dst in lin_descs(c):
                pltpu.async_copy(src, dst, lin_s)

        def wait_linear(c):
            for src, dst in lin_descs(c):
                pltpu.make_async_copy(src, dst, lin_s).wait()

        def gat_descs(c):
            oi = (c % 3) * CK
            o = (c % 2) * CK
            return [
                (h_hbm.at[idx_v.at[pl.ds(oi + g * 128, 128)]],
                 rows_v.at[pl.ds(o + g * 128, 128)])
                for g in range(G)
            ]

        def out_desc(c, par):
            return (out_v.at[pl.ds(par * C, C)],
                    out_hbm.at[pl.ds(base0 + c * C, C)])

        def compute(c):
            o = (c % 2) * CK
            ol = (c % 3) * CK
            oc = (c % 3) * C
            oo = (c % 2) * C

            def node_body(i, carry2):
                wvec = jnp.exp(dsq_v[pl.ds(ol + i * K, K)] * (-10.0))
                rb = o + i * K
                s = [jnp.zeros((LANES,), jnp.float32) for _ in range(HV)]
                m = [jnp.full((LANES,), -jnp.inf, jnp.float32)
                     for _ in range(HV)]
                for k in range(K):
                    wk = wvec[k]
                    for j in range(HV):
                        wg = rows_v[rb + k, pl.ds(j * LANES, LANES)] * wk
                        s[j] = s[j] + wg
                        m[j] = jnp.maximum(m[j], wg)
                for j in range(HV):
                    ow = own_v[oc + i, pl.ds(j * LANES, LANES)]
                    out_v[oo + i, pl.ds(j * LANES, LANES)] = (
                        s[j] * (1.0 / K) - ow)
                    out_v[oo + i, pl.ds(H + j * LANES, LANES)] = m[j] - ow
                return carry2

            lax.fori_loop(0, C, node_body, 0)

        issue_linear(0)
        wait_linear(0)
        for src, dst in gat_descs(0):
            pltpu.async_copy(src, dst, gat_s)
        issue_linear(1)

        def chunk_body(c, carry):
            @pl.when(c < n_chunks - 1)
            def _():
                wait_linear(c + 1)
            for src, dst in gat_descs(c):
                pltpu.make_async_copy(src, dst, gat_s).wait()

            @pl.when(c < n_chunks - 1)
            def _():
                for src, dst in gat_descs(c + 1):
                    pltpu.async_copy(src, dst, gat_s)

            @pl.when(c < n_chunks - 2)
            def _():
                issue_linear(c + 2)

            for par, sem in ((0, out_s0), (1, out_s1)):
                @pl.when(jnp.logical_and(c >= 2, c % 2 == par))
                def _(par=par, sem=sem):
                    src, dst = out_desc(c - 2, par)
                    pltpu.make_async_copy(src, dst, sem).wait()

            compute(c)

            for par, sem in ((0, out_s0), (1, out_s1)):
                @pl.when(c % 2 == par)
                def _(par=par, sem=sem):
                    src, dst = out_desc(c, par)
                    pltpu.async_copy(src, dst, sem)
            return carry

        lax.fori_loop(0, n_chunks, chunk_body, 0)
        for c in (n_chunks - 2, n_chunks - 1):
            src, dst = out_desc(c, c % 2)
            pltpu.make_async_copy(src, dst, out_s0 if c % 2 == 0 else out_s1).wait()

    return acc


def kernel(x, neighbor_indices, distancesq, W0, b0, W1, b1):
    n, d = x.shape
    K = neighbor_indices.shape[1]
    H = W0.shape[1]

    C = 32
    per_w = -(-n // (NW * 2 * C)) * 2 * C
    n_pad = per_w * NW

    pad_n = n_pad - n
    x_pad = jnp.pad(x, ((0, pad_n), (0, 0)))
    idx_flat = jnp.pad(neighbor_indices, ((0, pad_n), (0, 0))).reshape(-1)
    dsq_flat = jnp.pad(distancesq, ((0, pad_n), (0, 0))).reshape(-1)

    acc = _make_accumulate(n_pad, K, H, per_w, C)

    h0 = _dense_relu(x_pad, W0, b0, block_rows=512)
    f1 = acc(h0, idx_flat, dsq_flat)
    h1 = _dense_relu(f1, W1, b1, block_rows=512)
    f2 = acc(h1, idx_flat, dsq_flat)
    return jnp.concatenate([f1[:n], f2[:n], x], axis=-1)
